# baseline (speedup 1.0000x reference)
import functools

import jax
import jax.numpy as jnp
from jax import lax
from jax.experimental import pallas as pl
from jax.experimental.pallas import tpu as pltpu
from jax.experimental.pallas import tpu_sc as plsc

N_TILE = 2048
K_TILE = 2048
LANES = 16


def _residual_kernel(a_ref, x_ref, y_ref, r_ref):
    k = pl.program_id(0)
    part = jax.lax.dot(a_ref[...], x_ref[...])

    @pl.when(k == 0)
    def _():
        r_ref[...] = y_ref[...].T - part

    @pl.when(k != 0)
    def _():
        r_ref[...] = r_ref[...] - part


def _corr_kernel(a_ref, ry_ref, val_ref, idx_ref, g_ref, d_ref):
    n = pl.program_id(0)
    a = a_ref[...]
    cg = jax.lax.dot_general(a, ry_ref[...], (((0,), (0,)), ((), ())))
    b = cg.shape[1] // 2
    corr = jnp.abs(cg[:, :b])
    gmat = cg[:, b:]

    m = jnp.max(corr, axis=0, keepdims=True)
    rows = jax.lax.broadcasted_iota(jnp.int32, corr.shape, 0)
    big = jnp.int32(corr.shape[0])
    loc = jnp.min(
        jnp.where(corr == m, rows, big), axis=0, keepdims=True
    )
    sel = rows == loc
    gsel = jnp.sum(jnp.where(sel, gmat, 0.0), axis=0, keepdims=True)
    dcol = jnp.sum(a * a, axis=0, keepdims=True)
    dsel = jnp.sum(jnp.where(sel, dcol.T, 0.0), axis=0, keepdims=True)

    val_ref[...] = m[None]
    idx_ref[...] = (loc + n * a.shape[1])[None]
    g_ref[...] = gsel[None]
    d_ref[...] = dsel[None]


def _mask_kernel(val_ref, idx_ref, s_ref, so_ref):
    n = pl.program_id(0)
    v = val_ref[:, 0, :]
    m = jnp.max(v, axis=0, keepdims=True)
    tiles = jax.lax.broadcasted_iota(jnp.int32, v.shape, 0)
    big = jnp.int32(v.shape[0])
    wt = jnp.min(jnp.where(v == m, tiles, big), axis=0, keepdims=True)
    sel = tiles == wt
    j = jnp.sum(jnp.where(sel, idx_ref[:, 0, :], 0), axis=0, keepdims=True)
    rows = jax.lax.broadcasted_iota(jnp.int32, so_ref.shape, 0) + n * so_ref.shape[0]
    so_ref[...] = (rows == j) | s_ref[...]


def _sc_scatter_body(
    n_tiles, n_rows, batch, n_workers,
    vals_h, idxs_h, gs_h, ds_h, h_out,
    vals_v, idxs_v, gs_v, ds_v, hbuf,
):
    num_cores = 2
    wid = lax.axis_index("s") * num_cores + lax.axis_index("c")
    rows = n_rows // n_workers
    base = wid * rows

    pltpu.sync_copy(vals_h, vals_v)
    pltpu.sync_copy(idxs_h, idxs_v)
    pltpu.sync_copy(gs_h, gs_v)
    pltpu.sync_copy(ds_h, ds_v)

    merged = []
    for g in range(batch // LANES):
        m = vals_v[pl.ds(g * LANES, LANES)]
        j = idxs_v[pl.ds(g * LANES, LANES)]
        gg = gs_v[pl.ds(g * LANES, LANES)]
        dd = ds_v[pl.ds(g * LANES, LANES)]
        for t in range(1, n_tiles):
            off = t * batch + g * LANES
            v = vals_v[pl.ds(off, LANES)]
            gt = v > m
            m = jnp.where(gt, v, m)
            j = jnp.where(gt, idxs_v[pl.ds(off, LANES)], j)
            gg = jnp.where(gt, gs_v[pl.ds(off, LANES)], gg)
            dd = jnp.where(gt, ds_v[pl.ds(off, LANES)], dd)
        x = gg / dd
        merged.append(j)
        merged.append(x)

    def _fill_body(r, carry):
        rv = base + r
        for g in range(batch // LANES):
            j_g = carry[2 * g]
            x_g = carry[2 * g + 1]
            hbuf[pl.ds(pl.multiple_of(r * batch + g * LANES, LANES), LANES)] = (
                jnp.where(j_g == rv, x_g, 0.0)
            )
        return carry

    lax.fori_loop(0, rows, _fill_body, tuple(merged))

    pltpu.sync_copy(hbuf, h_out.at[pl.ds(base * batch, rows * batch)])


def kernel(X, Y, S, A):
    M, N = A.shape
    B = X.shape[1]
    n_k = N // K_TILE
    n_n = N // N_TILE

    R = pl.pallas_call(
        _residual_kernel,
        grid=(n_k,),
        in_specs=[
            pl.BlockSpec((M, K_TILE), lambda k: (0, k)),
            pl.BlockSpec((K_TILE, B), lambda k: (k, 0)),
            pl.BlockSpec((B, M), lambda k: (0, 0)),
        ],
        out_specs=pl.BlockSpec((M, B), lambda k: (0, 0)),
        out_shape=jax.ShapeDtypeStruct((M, B), A.dtype),
    )(A, X, Y)

    RY = jnp.concatenate([R, Y.T], axis=1)

    stat_shape = jax.ShapeDtypeStruct((n_n, 1, B), jnp.float32)
    stat_spec = pl.BlockSpec((1, 1, B), lambda n: (n, 0, 0))
    vals, idxs, gs, ds = pl.pallas_call(
        _corr_kernel,
        grid=(n_n,),
        in_specs=[
            pl.BlockSpec((M, N_TILE), lambda n: (0, n)),
            pl.BlockSpec((M, 2 * B), lambda n: (0, 0)),
        ],
        out_specs=[stat_spec, stat_spec, stat_spec, stat_spec],
        out_shape=[
            stat_shape,
            jax.ShapeDtypeStruct((n_n, 1, B), jnp.int32),
            stat_shape,
            stat_shape,
        ],
    )(A, RY)

    n_workers = 32
    sc_scatter = pl.kernel(
        functools.partial(_sc_scatter_body, n_n, N, B, n_workers),
        out_type=jax.ShapeDtypeStruct((N * B,), jnp.float32),
        mesh=plsc.VectorSubcoreMesh(core_axis_name="c", subcore_axis_name="s"),
        scratch_types=[
            pltpu.VMEM((n_n * B,), jnp.float32),
            pltpu.VMEM((n_n * B,), jnp.int32),
            pltpu.VMEM((n_n * B,), jnp.float32),
            pltpu.VMEM((n_n * B,), jnp.float32),
            pltpu.VMEM((N // n_workers * B,), jnp.float32),
        ],
    )

    H_flat = sc_scatter(
        jnp.reshape(vals, (n_n * B,)),
        jnp.reshape(idxs, (n_n * B,)),
        jnp.reshape(gs, (n_n * B,)),
        jnp.reshape(ds, (n_n * B,)),
    )
    H = jnp.reshape(H_flat, (N, B))

    full_spec = pl.BlockSpec((n_n, 1, B), lambda n: (0, 0, 0))
    S_out = pl.pallas_call(
        _mask_kernel,
        grid=(n_n,),
        in_specs=[
            full_spec,
            full_spec,
            pl.BlockSpec((N_TILE, B), lambda n: (n, 0)),
        ],
        out_specs=pl.BlockSpec((N_TILE, B), lambda n: (n, 0)),
        out_shape=jax.ShapeDtypeStruct((N, B), jnp.bool_),
    )(vals, idxs, S)

    return (H, S_out)
